# baseline (speedup 1.0000x reference)
import functools

import jax
import jax.numpy as jnp
from jax import lax
from jax.experimental import pallas as pl
from jax.experimental.pallas import tpu as pltpu
from jax.experimental.pallas import tpu_sc as plsc

NC = 2
NS = 16
NW = NC * NS
LCH = 128
K2 = 2
NSLOT = 2
IGRP = 8
ROWS_G = IGRP * K2


def _ceil_to(a, m):
  return (a + m - 1) // m * m


def _tc_node_matmuls(x, wl, wr):
  n, _ = x.shape
  dh = wl.shape[1]

  def body(x_ref, wl_ref, wr_ref, t_ref):
    xv = x_ref[...]
    t = jnp.concatenate(
        [jnp.dot(xv, wl_ref[...], preferred_element_type=jnp.float32),
         jnp.dot(xv, wr_ref[...], preferred_element_type=jnp.float32)],
        axis=1)
    t_ref[0] = t
    t_ref[1] = t

  return pl.pallas_call(
      body,
      out_shape=jax.ShapeDtypeStruct((NC, n, 2 * dh), jnp.float32),
  )(x, wl, wr)


def _tc_edge(ef, gg, we, att2, e_real, d, off1, off2, ch=8192):
  _, e_pad, wg = gg.shape
  de = ef.shape[1]
  ns = d // 64
  grid = e_pad // ch

  def body(ef_ref, g1_ref, g2_ref, we_ref, att_ref, out_ref):
    i = pl.program_id(0)
    g1v = g1_ref[0][:, off1:off1 + d]
    g2v = g2_ref[0][:, off2:off2 + d]
    ew = jnp.dot(ef_ref[...], we_ref[...], preferred_element_type=jnp.float32)
    m = g1v + g2v + ew
    e = jnp.where(m >= 0.0, m, 0.2 * m)
    logit = jnp.sum(e * att_ref[...], axis=1, keepdims=True)
    ids = lax.broadcasted_iota(jnp.int32, (ch, 1), 0) + i * ch
    ex = jnp.where(ids < e_real, jnp.exp(logit), 0.0)
    pad = jnp.zeros((ch, 63), jnp.float32)
    slabs = [
        jnp.concatenate([ex * g1v[:, 64 * k:64 * (k + 1)], ex, pad], axis=1)
        for k in range(ns)
    ]
    out_ref[...] = jnp.stack(slabs)

  return pl.pallas_call(
      body,
      grid=(grid,),
      in_specs=[
          pl.BlockSpec((ch, de), lambda i: (i, 0)),
          pl.BlockSpec((1, ch, wg), lambda i: (0, i, 0)),
          pl.BlockSpec((1, ch, wg), lambda i: (1, i, 0)),
          pl.BlockSpec((de, d), lambda i: (0, 0)),
          pl.BlockSpec((1, d), lambda i: (0, 0)),
      ],
      out_specs=pl.BlockSpec((ns, ch, 128), lambda i: (0, i, 0)),
      out_shape=jax.ShapeDtypeStruct((ns, e_pad, 128), jnp.float32),
  )(ef, gg, gg, we, att2)


def _tc_latent(accs, b2, mu_w, mu_b2, lv_w, lv_b2, eps, dwl, dwr):
  n = eps.shape[0]
  dh = mu_w.shape[1]
  d = dh
  dout = dwl.shape[1]

  na = len(accs)

  def body(*refs):
    acc_refs = refs[:na]
    (b_ref, muw_ref, mub_ref, lvw_ref, lvb_ref, eps_ref, dwl_ref,
     dwr_ref, z_ref, zlr_ref) = refs[na:]
    a = sum(r[0, :n] + r[1, :n] for r in acc_refs)
    red = a[:, :d] / (a[:, d:d + 1] + 1e-16) + b_ref[...]
    mu = jnp.dot(red, muw_ref[...], preferred_element_type=jnp.float32) + mub_ref[...]
    lv = jnp.dot(red, lvw_ref[...], preferred_element_type=jnp.float32) + lvb_ref[...]
    z = mu + 0.5 * jnp.exp(lv) * eps_ref[...]
    z_ref[...] = z
    zlr_ref[0] = jnp.dot(z, dwl_ref[...], preferred_element_type=jnp.float32)
    zlr_ref[1] = jnp.dot(z, dwr_ref[...], preferred_element_type=jnp.float32)

  return pl.pallas_call(
      body,
      out_shape=(
          jax.ShapeDtypeStruct((n, dh), jnp.float32),
          jax.ShapeDtypeStruct((NC, n, dout), jnp.float32),
      ),
  )(*accs, b2, mu_w, mu_b2, lv_w, lv_b2, eps, dwl, dwr)


def _tc_final(accs, b2, n):
  na = len(accs)

  def body(*refs):
    acc_refs = refs[:na]
    b_ref, out_ref = refs[na:]
    halves = []
    for c in range(NC):
      a = sum(r[c, :n] for r in acc_refs)
      halves.append(a[:, :64] / (a[:, 64:65] + 1e-16))
    out_ref[...] = jnp.concatenate(halves, axis=1) + b_ref[...]

  return pl.pallas_call(
      body,
      out_shape=jax.ShapeDtypeStruct((n, 64 * NC), jnp.float32),
  )(*accs, b2)


def _sc_gather2(tbl2, idx2):
  _, n, d = tbl2.shape
  _, n_rows, _ = idx2.shape
  e_pad = n_rows * LCH
  per_w = e_pad // NS
  n_it = per_w // LCH
  n_grp = n_it // IGRP
  nslot = 4
  mesh = plsc.VectorSubcoreMesh(core_axis_name="c", subcore_axis_name="s",
                                num_cores=NC, num_subcores=NS)

  @functools.partial(
      pl.kernel,
      out_type=jax.ShapeDtypeStruct((NC, e_pad, d), jnp.float32),
      mesh=mesh,
      scratch_types=[
          pltpu.VMEM((2, IGRP, LCH), jnp.int32),
          pltpu.VMEM((nslot, LCH, d), jnp.float32),
      ] + [pltpu.SemaphoreType.DMA] * (2 * nslot),
  )
  def k(t_h, i_h, g_h, iv, rv, sg0, sg1, sg2, sg3, sw0, sw1, sw2, sw3):
    c = lax.axis_index("c")
    s = lax.axis_index("s")
    tbl = t_h.at[c]
    idx_h = i_h.at[c]
    out = g_h.at[c]
    base = s * per_w
    brow = s * n_it
    sg = (sg0, sg1, sg2, sg3)
    sw = (sw0, sw1, sw2, sw3)

    def launch(gslot, lk, b):
      pltpu.async_copy(tbl.at[iv.at[gslot, lk]], rv.at[b], sg[b])

    def gather_wait(b):
      pltpu.make_async_copy(tbl.at[iv.at[0, 0]], rv.at[b], sg[b]).wait()

    def wb_wait(b):
      pltpu.make_async_copy(rv.at[b], out.at[pl.ds(base, LCH)],
                            sw[b]).wait()

    pltpu.sync_copy(idx_h.at[pl.ds(brow, IGRP)], iv.at[0])
    launch(0, 0, 0)
    launch(0, 1, 1)

    def body(g, carry):
      gs = lax.rem(g, 2)
      gs1 = lax.rem(g + 1, 2)

      @pl.when(g < n_grp - 1)
      def _():
        pltpu.sync_copy(idx_h.at[pl.ds(brow + (g + 1) * IGRP, IGRP)],
                        iv.at[gs1])

      for kk in range(IGRP):
        b = kk % nslot
        b2 = (kk + 2) % nslot
        off = base + (g * IGRP + kk) * LCH
        gather_wait(b)
        pltpu.async_copy(rv.at[b], out.at[pl.ds(off, LCH)], sw[b])
        if kk < 2:
          @pl.when(g > 0)
          def _():
            wb_wait(b2)
        else:
          wb_wait(b2)
        if kk < IGRP - 2:
          launch(gs, kk + 2, b2)
        else:
          @pl.when(g < n_grp - 1)
          def _():
            launch(gs1, kk + 2 - IGRP, b2)
      return carry

    lax.fori_loop(0, n_grp, body, 0)
    wb_wait((n_it - 2) % nslot)
    wb_wait((n_it - 1) % nslot)

  return k(tbl2, idx2)


def _sc_scatter_add(rows3, dst, zeros_init):
  nslab, e_pad, dext = rows3.shape
  n_pad = zeros_init.shape[0]
  nworker = NW if nslab == 1 else NS
  per_w = e_pad // nworker
  igrp = 8
  rows_g = igrp
  n_it = per_w // LCH
  n_grp = n_it // igrp
  n_rows_s = n_pad // NS
  mesh = plsc.VectorSubcoreMesh(core_axis_name="c", subcore_axis_name="s",
                                num_cores=NC, num_subcores=NS)

  @functools.partial(
      pl.kernel,
      out_type=jax.ShapeDtypeStruct((NC, n_pad, dext), jnp.float32),
      mesh=mesh,
      scratch_types=[
          pltpu.VMEM((2, rows_g, LCH), jnp.int32),
          pltpu.VMEM((NSLOT, LCH, dext), jnp.float32),
          pltpu.VMEM_SHARED((n_pad, dext), jnp.float32),
      ] + [pltpu.SemaphoreType.DMA] * (2 * NSLOT),
  )
  def k(rows_h, dst_h, zer_h, out_h, iv, rv, acc, sr0, sr1, ss0, ss1):
    c = lax.axis_index("c")
    s = lax.axis_index("s")
    if nslab == 1:
      slab = rows_h.at[0]
      wid = s * NC + c
    else:
      slab = rows_h.at[c]
      wid = s
    base = wid * per_w
    brow = wid * (per_w // LCH)
    sr = (sr0, sr1)
    ss = (ss0, ss1)
    pltpu.sync_copy(zer_h.at[pl.ds(s * n_rows_s, n_rows_s)],
                    acc.at[pl.ds(s * n_rows_s, n_rows_s)])

    def load(i, b):
      pltpu.async_copy(slab.at[pl.ds(base + i * LCH, LCH)], rv.at[b], sr[b])

    def load_wait(b):
      pltpu.make_async_copy(slab.at[pl.ds(base, LCH)], rv.at[b],
                            sr[b]).wait()

    def scat_wait(b):
      pltpu.make_async_copy(rv.at[b], acc.at[iv.at[0, 0]], ss[b]).wait()

    pltpu.sync_copy(dst_h.at[pl.ds(brow, rows_g)], iv.at[0])
    load(0, 0)
    load(1, 1)
    plsc.subcore_barrier()

    def body(g, carry):
      gs = lax.rem(g, 2)
      gs1 = lax.rem(g + 1, 2)

      @pl.when(g < n_grp - 1)
      def _():
        pltpu.sync_copy(dst_h.at[pl.ds(brow + (g + 1) * rows_g, rows_g)],
                        iv.at[gs1])

      for kk in range(igrp):
        b = kk % NSLOT
        load_wait(b)
        pltpu.async_copy(rv.at[b], acc.at[iv.at[gs, kk]], ss[b], add=True)
        scat_wait(b)
        if kk < igrp - 2:
          load(g * igrp + kk + 2, b)
        else:
          @pl.when(g < n_grp - 1)
          def _():
            load(g * igrp + kk + 2, b)
      return carry

    lax.fori_loop(0, n_grp, body, 0)
    plsc.subcore_barrier()
    pltpu.sync_copy(acc.at[pl.ds(s * n_rows_s, n_rows_s)],
                    out_h.at[c, pl.ds(s * n_rows_s, n_rows_s)])

  return k(rows3, dst, zeros_init)


def _gat_layer(tbl2, idx2, dst2, ef_p, we, att2, e_real, zeros_init,
               d, off1, off2):
  gg = _sc_gather2(tbl2, idx2)
  rows = _tc_edge(ef_p, gg, we, att2, e_real, d, off1, off2)
  return _sc_scatter_add(rows, dst2, zeros_init)


def kernel(x, edge_index, edge_feat, enc_Wl, enc_Wr, enc_We, enc_att, enc_b,
           mu_W, mu_b, lv_W, lv_b, dec_Wl, dec_Wr, dec_We, dec_att, dec_b):
  n, _ = x.shape
  e = edge_index.shape[1]
  de = edge_feat.shape[1]
  dh = enc_Wl.shape[1]
  dout = dec_Wl.shape[1]

  e_pad = _ceil_to(e, NW * LCH * K2 * IGRP)
  pad = e_pad - e
  pad_idx = (jnp.arange(pad, dtype=jnp.int32) * 37) % n
  src_p = jnp.concatenate([edge_index[0], pad_idx])
  dst_p = jnp.concatenate([edge_index[1], pad_idx])
  ef_p = jnp.concatenate([edge_feat, jnp.zeros((pad, de), jnp.float32)])
  idx2 = jnp.stack([src_p, dst_p]).reshape(2, e_pad // LCH, LCH)
  dst2 = dst_p.reshape(e_pad // LCH, LCH)
  nh = 1
  e_half = e_pad // nh
  h_rows = e_half // LCH
  halves = []
  for h in range(nh):
    halves.append((
        idx2[:, h * h_rows:(h + 1) * h_rows],
        dst2[h * h_rows:(h + 1) * h_rows],
        ef_p[h * e_half:(h + 1) * e_half],
        e - h * e_half,
    ))

  n_pad = _ceil_to(n, 8 * NS)
  zeros_acc = jnp.zeros((n_pad, 128), jnp.float32)
  eps = jax.random.normal(jax.random.key(42), (n, dh), dtype=jnp.float32)

  t_enc = _tc_node_matmuls(x, enc_Wl, enc_Wr)
  acc1 = [
      _gat_layer(t_enc, hi, hd, hef, enc_We, enc_att.reshape(1, dh),
                 her, zeros_acc, dh, 0, dh)
      for hi, hd, hef, her in halves
  ]

  z, zlr = _tc_latent(acc1, enc_b.reshape(1, dh), mu_W,
                      mu_b.reshape(1, dh), lv_W, lv_b.reshape(1, dh),
                      eps, dec_Wl, dec_Wr)

  acc2 = [
      _gat_layer(zlr, hi, hd, hef, dec_We, dec_att.reshape(1, dout),
                 her, zeros_acc, dout, 0, 0)
      for hi, hd, hef, her in halves
  ]
  recon = _tc_final(acc2, dec_b.reshape(1, dout), n)
  return (z, recon)


# speedup vs baseline: 1.0188x; 1.0188x over previous
import functools

import jax
import jax.numpy as jnp
from jax import lax
from jax.experimental import pallas as pl
from jax.experimental.pallas import tpu as pltpu
from jax.experimental.pallas import tpu_sc as plsc

NC = 2
NS = 16
NW = NC * NS
LCH = 128
K2 = 2
NSLOT = 2
IGRP = 8
ROWS_G = IGRP * K2


def _ceil_to(a, m):
  return (a + m - 1) // m * m


def _tc_node_matmuls(x, wl, wr):
  n, _ = x.shape
  dh = wl.shape[1]

  def body(x_ref, wl_ref, wr_ref, t_ref):
    xv = x_ref[...]
    t = jnp.concatenate(
        [jnp.dot(xv, wl_ref[...], preferred_element_type=jnp.float32),
         jnp.dot(xv, wr_ref[...], preferred_element_type=jnp.float32)],
        axis=1)
    t_ref[0] = t
    t_ref[1] = t

  return pl.pallas_call(
      body,
      out_shape=jax.ShapeDtypeStruct((NC, n, 2 * dh), jnp.float32),
  )(x, wl, wr)


def _tc_edge(ef, gg, we, att2, e_real, d, off1, off2, ch=8192):
  _, e_pad, wg = gg.shape
  de = ef.shape[1]
  ns = d // 64
  grid = e_pad // ch

  def body(ef_ref, g1_ref, g2_ref, we_ref, att_ref, out_ref):
    i = pl.program_id(0)
    g1v = g1_ref[0][:, off1:off1 + d]
    g2v = g2_ref[0][:, off2:off2 + d]
    ew = jnp.dot(ef_ref[...], we_ref[...], preferred_element_type=jnp.float32)
    m = g1v + g2v + ew
    e = jnp.where(m >= 0.0, m, 0.2 * m)
    logit = jnp.sum(e * att_ref[...], axis=1, keepdims=True)
    ids = lax.broadcasted_iota(jnp.int32, (ch, 1), 0) + i * ch
    ex = jnp.where(ids < e_real, jnp.exp(logit), 0.0)
    pad = jnp.zeros((ch, 63), jnp.float32)
    slabs = [
        jnp.concatenate([ex * g1v[:, 64 * k:64 * (k + 1)], ex, pad], axis=1)
        for k in range(ns)
    ]
    out_ref[...] = jnp.stack(slabs)

  return pl.pallas_call(
      body,
      grid=(grid,),
      in_specs=[
          pl.BlockSpec((ch, de), lambda i: (i, 0)),
          pl.BlockSpec((1, ch, wg), lambda i: (0, i, 0)),
          pl.BlockSpec((1, ch, wg), lambda i: (1, i, 0)),
          pl.BlockSpec((de, d), lambda i: (0, 0)),
          pl.BlockSpec((1, d), lambda i: (0, 0)),
      ],
      out_specs=pl.BlockSpec((ns, ch, 128), lambda i: (0, i, 0)),
      out_shape=jax.ShapeDtypeStruct((ns, e_pad, 128), jnp.float32),
  )(ef, gg, gg, we, att2)


def _tc_latent(accs, b2, mu_w, mu_b2, lv_w, lv_b2, eps, dwl, dwr):
  n = eps.shape[0]
  dh = mu_w.shape[1]
  d = dh
  dout = dwl.shape[1]

  na = len(accs)

  def body(*refs):
    acc_refs = refs[:na]
    (b_ref, muw_ref, mub_ref, lvw_ref, lvb_ref, eps_ref, dwl_ref,
     dwr_ref, z_ref, zlr_ref) = refs[na:]
    a = sum(r[0, :n] + r[1, :n] for r in acc_refs)
    red = a[:, :d] / (a[:, d:d + 1] + 1e-16) + b_ref[...]
    mu = jnp.dot(red, muw_ref[...], preferred_element_type=jnp.float32) + mub_ref[...]
    lv = jnp.dot(red, lvw_ref[...], preferred_element_type=jnp.float32) + lvb_ref[...]
    z = mu + 0.5 * jnp.exp(lv) * eps_ref[...]
    z_ref[...] = z
    zlr_ref[0] = jnp.dot(z, dwl_ref[...], preferred_element_type=jnp.float32)
    zlr_ref[1] = jnp.dot(z, dwr_ref[...], preferred_element_type=jnp.float32)

  return pl.pallas_call(
      body,
      out_shape=(
          jax.ShapeDtypeStruct((n, dh), jnp.float32),
          jax.ShapeDtypeStruct((NC, n, dout), jnp.float32),
      ),
  )(*accs, b2, mu_w, mu_b2, lv_w, lv_b2, eps, dwl, dwr)


def _tc_final(accs, b2, n):
  na = len(accs)

  def body(*refs):
    acc_refs = refs[:na]
    b_ref, out_ref = refs[na:]
    halves = []
    for c in range(NC):
      a = sum(r[c, :n] for r in acc_refs)
      halves.append(a[:, :64] / (a[:, 64:65] + 1e-16))
    out_ref[...] = jnp.concatenate(halves, axis=1) + b_ref[...]

  return pl.pallas_call(
      body,
      out_shape=jax.ShapeDtypeStruct((n, 64 * NC), jnp.float32),
  )(*accs, b2)


def _sc_gather2(tbl2, idx2):
  _, n, d = tbl2.shape
  _, n_rows, _ = idx2.shape
  e_pad = n_rows * LCH
  per_w = e_pad // NS
  n_it = per_w // LCH
  n_grp = n_it // IGRP
  nslot = 4
  mesh = plsc.VectorSubcoreMesh(core_axis_name="c", subcore_axis_name="s",
                                num_cores=NC, num_subcores=NS)

  @functools.partial(
      pl.kernel,
      out_type=jax.ShapeDtypeStruct((NC, e_pad, d), jnp.float32),
      mesh=mesh,
      scratch_types=[
          pltpu.VMEM((2, IGRP, LCH), jnp.int32),
          pltpu.VMEM((nslot, LCH, d), jnp.float32),
      ] + [pltpu.SemaphoreType.DMA] * (2 * nslot),
  )
  def k(t_h, i_h, g_h, iv, rv, sg0, sg1, sg2, sg3, sw0, sw1, sw2, sw3):
    c = lax.axis_index("c")
    s = lax.axis_index("s")
    tbl = t_h.at[c]
    idx_h = i_h.at[c]
    out = g_h.at[c]
    base = s * per_w
    brow = s * n_it
    sg = (sg0, sg1, sg2, sg3)
    sw = (sw0, sw1, sw2, sw3)

    def launch(gslot, lk, b):
      pltpu.async_copy(tbl.at[iv.at[gslot, lk]], rv.at[b], sg[b])

    def gather_wait(b):
      pltpu.make_async_copy(tbl.at[iv.at[0, 0]], rv.at[b], sg[b]).wait()

    def wb_wait(b):
      pltpu.make_async_copy(rv.at[b], out.at[pl.ds(base, LCH)],
                            sw[b]).wait()

    pltpu.sync_copy(idx_h.at[pl.ds(brow, IGRP)], iv.at[0])
    launch(0, 0, 0)
    launch(0, 1, 1)

    def body(g, carry):
      gs = lax.rem(g, 2)
      gs1 = lax.rem(g + 1, 2)

      @pl.when(g < n_grp - 1)
      def _():
        pltpu.sync_copy(idx_h.at[pl.ds(brow + (g + 1) * IGRP, IGRP)],
                        iv.at[gs1])

      for kk in range(IGRP):
        b = kk % nslot
        b2 = (kk + 2) % nslot
        off = base + (g * IGRP + kk) * LCH
        gather_wait(b)
        pltpu.async_copy(rv.at[b], out.at[pl.ds(off, LCH)], sw[b])
        if kk < 2:
          @pl.when(g > 0)
          def _():
            wb_wait(b2)
        else:
          wb_wait(b2)
        if kk < IGRP - 2:
          launch(gs, kk + 2, b2)
        else:
          @pl.when(g < n_grp - 1)
          def _():
            launch(gs1, kk + 2 - IGRP, b2)
      return carry

    lax.fori_loop(0, n_grp, body, 0)
    wb_wait((n_it - 2) % nslot)
    wb_wait((n_it - 1) % nslot)

  return k(tbl2, idx2)


def _sc_scatter_add(rows3, dst, zeros_init):
  nslab, e_pad, dext = rows3.shape
  n_pad = zeros_init.shape[0]
  nworker = NW if nslab == 1 else NS
  per_w = e_pad // nworker
  igrp = 8
  rows_g = igrp
  n_it = per_w // LCH
  n_grp = n_it // igrp
  n_rows_s = n_pad // NS
  mesh = plsc.VectorSubcoreMesh(core_axis_name="c", subcore_axis_name="s",
                                num_cores=NC, num_subcores=NS)

  @functools.partial(
      pl.kernel,
      out_type=jax.ShapeDtypeStruct((NC, n_pad, dext), jnp.float32),
      mesh=mesh,
      scratch_types=[
          pltpu.VMEM((2, rows_g, LCH), jnp.int32),
          pltpu.VMEM((NSLOT, LCH, dext), jnp.float32),
          pltpu.VMEM_SHARED((n_pad, dext), jnp.float32),
      ] + [pltpu.SemaphoreType.DMA] * (2 * NSLOT),
  )
  def k(rows_h, dst_h, zer_h, out_h, iv, rv, acc, sr0, sr1, ss0, ss1):
    c = lax.axis_index("c")
    s = lax.axis_index("s")
    if nslab == 1:
      slab = rows_h.at[0]
      wid = s * NC + c
    else:
      slab = rows_h.at[c]
      wid = s
    base = wid * per_w
    brow = wid * (per_w // LCH)
    sr = (sr0, sr1)
    ss = (ss0, ss1)
    pltpu.sync_copy(zer_h.at[pl.ds(s * n_rows_s, n_rows_s)],
                    acc.at[pl.ds(s * n_rows_s, n_rows_s)])

    def load(i, b):
      pltpu.async_copy(slab.at[pl.ds(base + i * LCH, LCH)], rv.at[b], sr[b])

    def load_wait(b):
      pltpu.make_async_copy(slab.at[pl.ds(base, LCH)], rv.at[b],
                            sr[b]).wait()

    def scat_wait(b):
      pltpu.make_async_copy(rv.at[b], acc.at[iv.at[0, 0]], ss[b]).wait()

    pltpu.sync_copy(dst_h.at[pl.ds(brow, rows_g)], iv.at[0])
    load(0, 0)
    load(1, 1)
    plsc.subcore_barrier()

    def body(g, carry):
      gs = lax.rem(g, 2)
      gs1 = lax.rem(g + 1, 2)

      @pl.when(g < n_grp - 1)
      def _():
        pltpu.sync_copy(dst_h.at[pl.ds(brow + (g + 1) * rows_g, rows_g)],
                        iv.at[gs1])

      for kk in range(igrp):
        b = kk % NSLOT
        load_wait(b)
        pltpu.async_copy(rv.at[b], acc.at[iv.at[gs, kk]], ss[b], add=True)
        scat_wait(b)
        if kk < igrp - 2:
          load(g * igrp + kk + 2, b)
        else:
          @pl.when(g < n_grp - 1)
          def _():
            load(g * igrp + kk + 2, b)
      return carry

    lax.fori_loop(0, n_grp, body, 0)
    plsc.subcore_barrier()
    pltpu.sync_copy(acc.at[pl.ds(s * n_rows_s, n_rows_s)],
                    out_h.at[c, pl.ds(s * n_rows_s, n_rows_s)])

  return k(rows3, dst, zeros_init)


def _gat_layer(tbl2, idx2, dst2, ef_p, we, att2, e_real, zeros_init,
               d, off1, off2):
  gg = _sc_gather2(tbl2, idx2)
  rows = _tc_edge(ef_p, gg, we, att2, e_real, d, off1, off2)
  return _sc_scatter_add(rows, dst2, zeros_init)


def kernel(x, edge_index, edge_feat, enc_Wl, enc_Wr, enc_We, enc_att, enc_b,
           mu_W, mu_b, lv_W, lv_b, dec_Wl, dec_Wr, dec_We, dec_att, dec_b):
  n, _ = x.shape
  e = edge_index.shape[1]
  de = edge_feat.shape[1]
  dh = enc_Wl.shape[1]
  dout = dec_Wl.shape[1]

  e_pad = _ceil_to(e, NW * LCH * K2 * IGRP)
  pad = e_pad - e
  pad_idx = (jnp.arange(pad, dtype=jnp.int32) * 37) % n
  src_p = jnp.concatenate([edge_index[0], pad_idx])
  dst_p = jnp.concatenate([edge_index[1], pad_idx])
  ef_p = jnp.concatenate([edge_feat, jnp.zeros((pad, de), jnp.float32)])
  idx2 = jnp.stack([src_p, dst_p]).reshape(2, e_pad // LCH, LCH)
  dst2 = dst_p.reshape(e_pad // LCH, LCH)
  nh = 2
  e_half = e_pad // nh
  h_rows = e_half // LCH
  halves = []
  for h in range(nh):
    halves.append((
        idx2[:, h * h_rows:(h + 1) * h_rows],
        dst2[h * h_rows:(h + 1) * h_rows],
        ef_p[h * e_half:(h + 1) * e_half],
        e - h * e_half,
    ))

  n_pad = _ceil_to(n, 8 * NS)
  zeros_acc = jnp.zeros((n_pad, 128), jnp.float32)
  eps = jax.random.normal(jax.random.key(42), (n, dh), dtype=jnp.float32)

  t_enc = _tc_node_matmuls(x, enc_Wl, enc_Wr)
  acc1 = [
      _gat_layer(t_enc, hi, hd, hef, enc_We, enc_att.reshape(1, dh),
                 her, zeros_acc, dh, 0, dh)
      for hi, hd, hef, her in halves
  ]

  z, zlr = _tc_latent(acc1, enc_b.reshape(1, dh), mu_W,
                      mu_b.reshape(1, dh), lv_W, lv_b.reshape(1, dh),
                      eps, dec_Wl, dec_Wr)

  acc2 = [
      _gat_layer(zlr, hi, hd, hef, dec_We, dec_att.reshape(1, dout),
                 her, zeros_acc, dout, 0, 0)
      for hi, hd, hef, her in halves
  ]
  recon = _tc_final(acc2, dec_b.reshape(1, dout), n)
  return (z, recon)
